# Initial kernel scaffold; baseline (speedup 1.0000x reference)
import jax
import jax.numpy as jnp
from jax.experimental import pallas as pl


def kernel(cls_scores, bbox_preds, priors, strides):
    raise NotImplementedError("write your pallas kernel here")


# speedup vs baseline: 23.6465x; 23.6465x over previous
import jax
import jax.numpy as jnp
from jax.experimental import pallas as pl

_PRE_TOP_K = 1000
_KEEP_TOP_K = 100
_IOU_THR = 0.65
_SCORE_THR = 0.25
_PAD = 1024


def _prep_body(cls_ref, bbox_ref, priors_ref, strides_ref, ms_ref, lb_ref, bx_ref):
    s = jax.nn.sigmoid(cls_ref[...])
    m = jnp.max(s, axis=1, keepdims=True)
    iota = jax.lax.broadcasted_iota(jnp.int32, s.shape, 1).astype(jnp.float32)
    lb = jnp.min(jnp.where(s >= m, iota, 3.4e38), axis=1, keepdims=True)
    ms_ref[...] = m
    lb_ref[...] = lb
    p = priors_ref[...]
    d = bbox_ref[...]
    st = strides_ref[...]
    tl = p - d[:, 0:2] * st
    br = p + d[:, 2:4] * st
    bx_ref[...] = jnp.concatenate([tl, br], axis=1)


def _nms_body(x1c_ref, y1c_ref, x2c_ref, y2c_ref,
              x1_ref, y1_ref, x2_ref, y2_ref, s_ref, out_ref):
    x1 = x1_ref[...]
    y1 = y1_ref[...]
    x2 = x2_ref[...]
    y2 = y2_ref[...]
    s = s_ref[...]
    area = jnp.maximum(x2 - x1, 0.0) * jnp.maximum(y2 - y1, 0.0)
    flat = (jax.lax.broadcasted_iota(jnp.int32, (8, 128), 0) * 128
            + jax.lax.broadcasted_iota(jnp.int32, (8, 128), 1))
    valid0 = jnp.where(s > _SCORE_THR, 1.0, 0.0)

    def body(i, valid):
        x1i = x1c_ref[pl.ds(i, 1), :]
        y1i = y1c_ref[pl.ds(i, 1), :]
        x2i = x2c_ref[pl.ds(i, 1), :]
        y2i = y2c_ref[pl.ds(i, 1), :]
        vi = jnp.sum(jnp.where(flat == i, valid, 0.0))
        iw = jnp.maximum(jnp.minimum(x2, x2i) - jnp.maximum(x1, x1i), 0.0)
        ih = jnp.maximum(jnp.minimum(y2, y2i) - jnp.maximum(y1, y1i), 0.0)
        inter = iw * ih
        areai = (jnp.maximum(x2i - x1i, 0.0) * jnp.maximum(y2i - y1i, 0.0))
        union = jnp.maximum(area + areai - inter, 1e-9)
        iou = inter / union
        kill = jnp.where((iou > _IOU_THR) & (flat != i), vi, 0.0)
        return valid * (1.0 - kill)

    valid = jax.lax.fori_loop(0, _PRE_TOP_K, body, valid0, unroll=False)
    out_ref[...] = jnp.where(valid > 0.0, s, -1.0)


def kernel(cls_scores, bbox_preds, priors, strides):
    B, N, C = cls_scores.shape
    R = 400
    ms, lb, bx = pl.pallas_call(
        _prep_body,
        grid=(N // R,),
        in_specs=[
            pl.BlockSpec((R, C), lambda i: (i, 0)),
            pl.BlockSpec((R, 4), lambda i: (i, 0)),
            pl.BlockSpec((R, 2), lambda i: (i, 0)),
            pl.BlockSpec((R, 1), lambda i: (i, 0)),
        ],
        out_specs=[
            pl.BlockSpec((R, 1), lambda i: (i, 0)),
            pl.BlockSpec((R, 1), lambda i: (i, 0)),
            pl.BlockSpec((R, 4), lambda i: (i, 0)),
        ],
        out_shape=[
            jax.ShapeDtypeStruct((N, 1), jnp.float32),
            jax.ShapeDtypeStruct((N, 1), jnp.float32),
            jax.ShapeDtypeStruct((N, 4), jnp.float32),
        ],
    )(cls_scores[0], bbox_preds[0], priors, strides[:, None])

    max_scores = ms[:, 0]
    labels = lb[:, 0]
    top_s, top_idx = jax.lax.top_k(max_scores, _PRE_TOP_K)
    boxes_sel = bx[top_idx]
    labels_sel = labels[top_idx]
    b_off = boxes_sel + labels_sel[:, None] * 4096.0

    pad = _PAD - _PRE_TOP_K
    bp = jnp.pad(b_off, ((0, pad), (0, 0)))
    coords = bp.reshape(8, 128, 4)
    s_pad = jnp.pad(top_s, (0, pad), constant_values=-1.0).reshape(8, 128)

    masked = pl.pallas_call(
        _nms_body,
        out_shape=jax.ShapeDtypeStruct((8, 128), jnp.float32),
    )(bp[:, 0:1], bp[:, 1:2], bp[:, 2:3], bp[:, 3:4],
      coords[..., 0], coords[..., 1], coords[..., 2], coords[..., 3], s_pad)

    masked = masked.reshape(_PAD)[:_PRE_TOP_K]
    top_sg, keep = jax.lax.top_k(masked, _KEEP_TOP_K)
    kv = top_sg > _SCORE_THR
    ob = jnp.where(kv[:, None], boxes_sel[keep], 0.0)
    osc = jnp.where(kv, top_s[keep], 0.0)
    ol = jnp.where(kv, labels_sel[keep], -1.0)
    dets = jnp.concatenate([ob, osc[:, None], ol[:, None]], axis=-1)
    return dets[None]


# speedup vs baseline: 24.4550x; 1.0342x over previous
import jax
import jax.numpy as jnp
from jax.experimental import pallas as pl

_PRE_TOP_K = 1000
_KEEP_TOP_K = 100
_IOU_THR = 0.65
_SCORE_THR = 0.25
_PAD = 1024


def _prep_body(cls_ref, bbox_ref, priors_ref, strides_ref, ms_ref, lb_ref, bx_ref):
    s = jax.nn.sigmoid(cls_ref[...])
    m = jnp.max(s, axis=1, keepdims=True)
    iota = jax.lax.broadcasted_iota(jnp.int32, s.shape, 1).astype(jnp.float32)
    lb = jnp.min(jnp.where(s >= m, iota, 3.4e38), axis=1, keepdims=True)
    ms_ref[...] = m
    lb_ref[...] = lb
    p = priors_ref[...]
    d = bbox_ref[...]
    st = strides_ref[...]
    tl = p - d[:, 0:2] * st
    br = p + d[:, 2:4] * st
    bx_ref[...] = jnp.concatenate([tl, br], axis=1)


def _nms_body(x1c_ref, y1c_ref, x2c_ref, y2c_ref,
              x1_ref, y1_ref, x2_ref, y2_ref, s_ref, out_ref):
    x1 = x1_ref[...]
    y1 = y1_ref[...]
    x2 = x2_ref[...]
    y2 = y2_ref[...]
    s = s_ref[...]
    area = jnp.maximum(x2 - x1, 0.0) * jnp.maximum(y2 - y1, 0.0)
    flat = (jax.lax.broadcasted_iota(jnp.int32, (8, 128), 0) * 128
            + jax.lax.broadcasted_iota(jnp.int32, (8, 128), 1))
    valid0 = jnp.where(s > _SCORE_THR, 1.0, 0.0)

    def body(i, valid):
        x1i = x1c_ref[pl.ds(i, 1), :]
        y1i = y1c_ref[pl.ds(i, 1), :]
        x2i = x2c_ref[pl.ds(i, 1), :]
        y2i = y2c_ref[pl.ds(i, 1), :]
        vi = jnp.sum(jnp.where(flat == i, valid, 0.0))
        iw = jnp.maximum(jnp.minimum(x2, x2i) - jnp.maximum(x1, x1i), 0.0)
        ih = jnp.maximum(jnp.minimum(y2, y2i) - jnp.maximum(y1, y1i), 0.0)
        inter = iw * ih
        areai = (jnp.maximum(x2i - x1i, 0.0) * jnp.maximum(y2i - y1i, 0.0))
        union = jnp.maximum(area + areai - inter, 1e-9)
        kill = jnp.where((inter > _IOU_THR * union) & (flat != i), vi, 0.0)
        return valid * (1.0 - kill)

    valid = jax.lax.fori_loop(0, _PRE_TOP_K, body, valid0, unroll=8)
    out_ref[...] = jnp.where(valid > 0.0, s, -1.0)


def kernel(cls_scores, bbox_preds, priors, strides):
    B, N, C = cls_scores.shape
    R = 400
    ms, lb, bx = pl.pallas_call(
        _prep_body,
        grid=(N // R,),
        in_specs=[
            pl.BlockSpec((R, C), lambda i: (i, 0)),
            pl.BlockSpec((R, 4), lambda i: (i, 0)),
            pl.BlockSpec((R, 2), lambda i: (i, 0)),
            pl.BlockSpec((R, 1), lambda i: (i, 0)),
        ],
        out_specs=[
            pl.BlockSpec((R, 1), lambda i: (i, 0)),
            pl.BlockSpec((R, 1), lambda i: (i, 0)),
            pl.BlockSpec((R, 4), lambda i: (i, 0)),
        ],
        out_shape=[
            jax.ShapeDtypeStruct((N, 1), jnp.float32),
            jax.ShapeDtypeStruct((N, 1), jnp.float32),
            jax.ShapeDtypeStruct((N, 4), jnp.float32),
        ],
    )(cls_scores[0], bbox_preds[0], priors, strides[:, None])

    max_scores = ms[:, 0]
    labels = lb[:, 0]
    top_s, top_idx = jax.lax.top_k(max_scores, _PRE_TOP_K)
    boxes_sel = bx[top_idx]
    labels_sel = labels[top_idx]
    b_off = boxes_sel + labels_sel[:, None] * 4096.0

    pad = _PAD - _PRE_TOP_K
    bp = jnp.pad(b_off, ((0, pad), (0, 0)))
    coords = bp.reshape(8, 128, 4)
    s_pad = jnp.pad(top_s, (0, pad), constant_values=-1.0).reshape(8, 128)

    masked = pl.pallas_call(
        _nms_body,
        out_shape=jax.ShapeDtypeStruct((8, 128), jnp.float32),
    )(bp[:, 0:1], bp[:, 1:2], bp[:, 2:3], bp[:, 3:4],
      coords[..., 0], coords[..., 1], coords[..., 2], coords[..., 3], s_pad)

    masked = masked.reshape(_PAD)[:_PRE_TOP_K]
    top_sg, keep = jax.lax.top_k(masked, _KEEP_TOP_K)
    kv = top_sg > _SCORE_THR
    ob = jnp.where(kv[:, None], boxes_sel[keep], 0.0)
    osc = jnp.where(kv, top_s[keep], 0.0)
    ol = jnp.where(kv, labels_sel[keep], -1.0)
    dets = jnp.concatenate([ob, osc[:, None], ol[:, None]], axis=-1)
    return dets[None]
